# Initial kernel scaffold; baseline (speedup 1.0000x reference)
import jax
import jax.numpy as jnp
from jax.experimental import pallas as pl


def kernel(x, edge_index, edge_weight, W1, b1, W2, b2):
    raise NotImplementedError("write your pallas kernel here")


# speedup vs baseline: 50.5108x; 50.5108x over previous
import functools

import jax
import jax.numpy as jnp
from jax import lax
from jax.experimental import pallas as pl
from jax.experimental.pallas import tpu as pltpu
from jax.experimental.pallas import tpu_sc as plsc

N = 10000
D = 128
H = 16
NC = 2
NS = 16
NW = NC * NS
L = 16

CHUNK = 128
NCHUNK = 81
EPW = NCHUNK * CHUNK
EPAD = NW * EPW
GROUPS = EPW // L
RPS = N // NS

_mesh = plsc.VectorSubcoreMesh(core_axis_name="c", subcore_axis_name="s")


@functools.partial(
    pl.kernel,
    out_type=jax.ShapeDtypeStruct((NW, N), jnp.float32),
    mesh=_mesh,
    compiler_params=pltpu.CompilerParams(needs_layout_passes=False, use_tc_tiling_on_sc=False),
    scratch_types=[
        pltpu.VMEM((GROUPS, L), jnp.int32),
        pltpu.VMEM((GROUPS, L), jnp.float32),
        pltpu.VMEM((N,), jnp.float32),
    ],
)
def _deg_kernel(col_hbm, ew_hbm, out_hbm, colbuf, ewbuf, acc):
    cid = lax.axis_index("c")
    sid = lax.axis_index("s")
    wid = cid * NS + sid

    @pl.loop(0, N // L)
    def _zero(i):
        acc[pl.ds(i * L, L)] = jnp.zeros((L,), jnp.float32)

    pltpu.sync_copy(col_hbm.at[wid], colbuf)
    pltpu.sync_copy(ew_hbm.at[wid], ewbuf)

    @pl.loop(0, GROUPS)
    def _body(g):
        cv = colbuf[g, :]
        wv = ewbuf[g, :]
        plsc.addupdate_scatter(acc, [cv], wv)

    pltpu.sync_copy(acc, out_hbm.at[wid])


@functools.partial(
    pl.kernel,
    out_type=jax.ShapeDtypeStruct((NC, N, H), jnp.float32),
    mesh=_mesh,
    compiler_params=pltpu.CompilerParams(needs_layout_passes=False, use_tc_tiling_on_sc=False),
    scratch_types=[
        pltpu.VMEM((NCHUNK, CHUNK), jnp.int32),
        pltpu.VMEM((NCHUNK, CHUNK), jnp.int32),
        pltpu.VMEM((NCHUNK, CHUNK), jnp.float32),
        pltpu.VMEM((CHUNK, H), jnp.float32),
        pltpu.VMEM((RPS, H), jnp.float32),
        pltpu.VMEM_SHARED((N, H), jnp.float32),
        pltpu.SemaphoreType.DMA,
    ],
)
def _agg1_kernel(row_hbm, col_hbm, ew_hbm, y_hbm, out_hbm,
                 rowbuf, colbuf, ewbuf, rows, bounce, acc, sem):
    cid = lax.axis_index("c")
    sid = lax.axis_index("s")
    wid = cid * NS + sid

    @pl.loop(0, RPS)
    def _zero(i):
        bounce[i, :] = jnp.zeros((H,), jnp.float32)

    pltpu.sync_copy(bounce, acc.at[pl.ds(sid * RPS, RPS)])
    pltpu.sync_copy(row_hbm.at[wid], rowbuf)
    pltpu.sync_copy(col_hbm.at[wid], colbuf)
    pltpu.sync_copy(ew_hbm.at[wid], ewbuf)
    plsc.subcore_barrier()

    @pl.loop(0, NCHUNK)
    def _chunk(c):
        pltpu.async_copy(y_hbm.at[rowbuf.at[c]], rows, sem).wait()

        @pl.loop(0, CHUNK // L)
        def _scale(g):
            wv = ewbuf[c, pl.ds(g * L, L)]
            base = g * L
            for e2 in range(L):
                rows[base + e2, :] = rows[base + e2, :] * wv[e2]

        pltpu.sync_copy(rows, acc.at[colbuf.at[c]], add=True)

    plsc.subcore_barrier()
    pltpu.sync_copy(acc.at[pl.ds(sid * RPS, RPS)], bounce)
    pltpu.sync_copy(bounce, out_hbm.at[cid, pl.ds(sid * RPS, RPS)])


@functools.partial(
    pl.kernel,
    out_type=jax.ShapeDtypeStruct((NW, N), jnp.float32),
    mesh=_mesh,
    compiler_params=pltpu.CompilerParams(needs_layout_passes=False, use_tc_tiling_on_sc=False),
    scratch_types=[
        pltpu.VMEM((GROUPS, L), jnp.int32),
        pltpu.VMEM((GROUPS, L), jnp.int32),
        pltpu.VMEM((GROUPS, L), jnp.float32),
        pltpu.VMEM((N,), jnp.float32),
        pltpu.VMEM((N,), jnp.float32),
    ],
)
def _agg2_kernel(row_hbm, col_hbm, ew_hbm, t_hbm, out_hbm,
                 rowbuf, colbuf, ewbuf, tbuf, acc):
    cid = lax.axis_index("c")
    sid = lax.axis_index("s")
    wid = cid * NS + sid

    @pl.loop(0, N // L)
    def _zero(i):
        acc[pl.ds(i * L, L)] = jnp.zeros((L,), jnp.float32)

    pltpu.sync_copy(t_hbm, tbuf)
    pltpu.sync_copy(row_hbm.at[wid], rowbuf)
    pltpu.sync_copy(col_hbm.at[wid], colbuf)
    pltpu.sync_copy(ew_hbm.at[wid], ewbuf)

    @pl.loop(0, GROUPS)
    def _body(g):
        rv = rowbuf[g, :]
        cv = colbuf[g, :]
        wv = ewbuf[g, :]
        tv = plsc.load_gather(tbuf, [rv])
        plsc.addupdate_scatter(acc, [cv], wv * tv)

    pltpu.sync_copy(acc, out_hbm.at[wid])


def _tc_prep_body(degp_ref, x_ref, w1_ref, dis_ref, y_ref):
    deg = jnp.sum(degp_ref[...], axis=0)
    dis = lax.rsqrt(deg)
    dis_ref[...] = dis
    xw = jnp.dot(x_ref[...], w1_ref[...], preferred_element_type=jnp.float32)
    y_ref[...] = xw * dis[:, None]


_tc_prep = pl.pallas_call(
    _tc_prep_body,
    out_shape=[
        jax.ShapeDtypeStruct((N,), jnp.float32),
        jax.ShapeDtypeStruct((N, H), jnp.float32),
    ],
)


def _tc_mid_body(p_ref, dis_ref, b1_ref, w2_ref, t_ref):
    p = p_ref[0] + p_ref[1]
    dis = dis_ref[...]
    h = jnp.maximum(p * dis[:, None] + b1_ref[...], 0.0)
    s = jnp.sum(h * w2_ref[...][:, 0], axis=1)
    t_ref[...] = s * dis


_tc_mid = pl.pallas_call(
    _tc_mid_body,
    out_shape=jax.ShapeDtypeStruct((N,), jnp.float32),
)


def _tc_fin_body(p_ref, dis_ref, b2_ref, out_ref):
    agg = jnp.sum(p_ref[...], axis=0)
    z = agg * dis_ref[...] + b2_ref[...]
    out_ref[...] = jax.nn.sigmoid(z)


_tc_fin = pl.pallas_call(
    _tc_fin_body,
    out_shape=jax.ShapeDtypeStruct((N,), jnp.float32),
)


def kernel(x, edge_index, edge_weight, W1, b1, W2, b2):
    e = edge_weight.shape[0]
    row = edge_index[0]
    col = edge_index[1]
    loop = jnp.arange(N, dtype=row.dtype)
    pad = EPAD - (e + N)
    zpad_i = jnp.zeros((pad,), dtype=row.dtype)
    row_a = jnp.concatenate([row, loop, zpad_i])
    col_a = jnp.concatenate([col, loop, zpad_i])
    ew_a = jnp.concatenate(
        [edge_weight, jnp.ones((N,), jnp.float32), jnp.zeros((pad,), jnp.float32)]
    )

    row_c = row_a.reshape(NW, NCHUNK, CHUNK)
    col_c = col_a.reshape(NW, NCHUNK, CHUNK)
    ew_c = ew_a.reshape(NW, NCHUNK, CHUNK)
    row_g = row_a.reshape(NW, GROUPS, L)
    col_g = col_a.reshape(NW, GROUPS, L)
    ew_g = ew_a.reshape(NW, GROUPS, L)

    degp = _deg_kernel(col_g, ew_g)
    dis, y = _tc_prep(degp, x, W1)
    p1 = _agg1_kernel(row_c, col_c, ew_c, y)
    t = _tc_mid(p1, dis, jnp.broadcast_to(b1, (1, H)), W2)
    p2 = _agg2_kernel(row_g, col_g, ew_g, t)
    out = _tc_fin(p2, dis, jnp.broadcast_to(b2, (1,)))
    return out
